# baseline (speedup 1.0000x reference)
import functools

import numpy as np
import jax
import jax.numpy as jnp
from jax import lax
from jax.experimental import pallas as pl
from jax.experimental.pallas import tpu as pltpu
from jax.experimental.pallas import tpu_sc as plsc

_B, _S, _DIN = 4, 512, 768
_C, _K = 2, 320
_ED = 128
_N = _B * _S


def _threefry2x32_np(k0, k1, x0, x1):
    rot = ((13, 15, 26, 6), (17, 29, 16, 24))
    ks = (np.uint32(k0), np.uint32(k1),
          np.uint32(k0) ^ np.uint32(k1) ^ np.uint32(0x1BD11BDA))
    x0 = (x0 + ks[0]).astype(np.uint32)
    x1 = (x1 + ks[1]).astype(np.uint32)
    for i in range(5):
        for r in rot[i % 2]:
            x0 = (x0 + x1).astype(np.uint32)
            x1 = ((x1 << np.uint32(r)) | (x1 >> np.uint32(32 - r))).astype(np.uint32)
            x1 = x1 ^ x0
        x0 = (x0 + ks[(i + 1) % 3]).astype(np.uint32)
        x1 = (x1 + ks[(i + 2) % 3] + np.uint32(i + 1)).astype(np.uint32)
    return x0, x1


def _gumbel_const():
    n = _N * _C * _K
    cnt = np.arange(n, dtype=np.uint32)
    h0, h1 = _threefry2x32_np(0, 42, np.zeros(n, np.uint32), cnt)
    bits = h0 ^ h1
    fb = (bits >> np.uint32(9)) | np.uint32(0x3F800000)
    tiny = np.float32(np.finfo(np.float32).tiny)
    floats = fb.view(np.float32) - np.float32(1.0)
    u = np.maximum(tiny, (floats * (np.float32(1.0) - tiny) + tiny).astype(np.float32))
    return (-np.log(-np.log(u))).astype(np.float32).reshape(_N, _C * _K)


_GUMBEL_T = np.ascontiguousarray(_gumbel_const().T)


def _tc_body(x_ref, w_ref, b_ref, g_ref, lt_ref, idx0_ref, idx1_ref):
    x = x_ref[...]
    w = w_ref[...]
    lt = lax.dot_general(
        w, x, dimension_numbers=(((1,), (1,)), ((), ())),
        preferred_element_type=jnp.float32,
    ) + b_ref[...][:, None]
    lt_ref[0] = lt
    noisy = lt + g_ref[...]
    for c, out_ref in ((0, idx0_ref), (1, idx1_ref)):
        v = noisy[c * _K:(c + 1) * _K, :]
        m = jnp.max(v, axis=0, keepdims=True)
        row = lax.broadcasted_iota(jnp.int32, v.shape, 0)
        out_ref[...] = jnp.min(
            jnp.where(v == m, row, jnp.int32(2**30)), axis=0) + c * _K


_GRID = 4
_CHUNK = _N // _GRID

_tc_call = pl.pallas_call(
    _tc_body,
    grid=(_GRID,),
    in_specs=[
        pl.BlockSpec((_CHUNK, _DIN), lambda i: (i, 0)),
        pl.BlockSpec((_C * _K, _DIN), lambda i: (0, 0)),
        pl.BlockSpec((_C * _K,), lambda i: (0,)),
        pl.BlockSpec((_C * _K, _CHUNK), lambda i: (0, i)),
    ],
    out_specs=(
        pl.BlockSpec((1, _C * _K, _CHUNK),
                     lambda i: (i // (_S // _CHUNK), 0, i % (_S // _CHUNK))),
        pl.BlockSpec((_CHUNK,), lambda i: (i,)),
        pl.BlockSpec((_CHUNK,), lambda i: (i,)),
    ),
    out_shape=(
        jax.ShapeDtypeStruct((_B, _C * _K, _S), jnp.float32),
        jax.ShapeDtypeStruct((_N,), jnp.int32),
        jax.ShapeDtypeStruct((_N,), jnp.int32),
    ),
)

_NC, _NS = 1, 16
_NW = _NC * _NS
_TPW = _N // _NW


_NSEG = 4
_SEG = _TPW // _NSEG


def _sc_gather_body(cb_hbm, idx0_hbm, idx1_hbm, out_hbm, idx_v, rows_v, gsem, wsem):
    wid = lax.axis_index("s") * _NC + lax.axis_index("c")
    base = wid * _TPW
    gathers = []
    for h in range(_NSEG):
        tb = base + h * _SEG
        lo = 2 * h * _SEG
        pltpu.sync_copy(idx0_hbm.at[pl.ds(tb, _SEG)], idx_v.at[pl.ds(lo, _SEG)])
        pltpu.sync_copy(idx1_hbm.at[pl.ds(tb, _SEG)], idx_v.at[pl.ds(lo + _SEG, _SEG)])
        gathers.append(pltpu.async_copy(
            cb_hbm.at[idx_v.at[pl.ds(lo, 2 * _SEG)]],
            rows_v.at[pl.ds(lo, 2 * _SEG)], gsem))
    writes = []
    for h in range(_NSEG):
        gathers[h].wait()
        tb = base + h * _SEG
        lo = 2 * h * _SEG
        writes.append(pltpu.async_copy(
            rows_v.at[pl.ds(lo, _SEG)],
            out_hbm.at[pl.ds(tb, _SEG), pl.ds(0, _ED)], wsem))
        writes.append(pltpu.async_copy(
            rows_v.at[pl.ds(lo + _SEG, _SEG)],
            out_hbm.at[pl.ds(tb, _SEG), pl.ds(_ED, _ED)], wsem))
    for w_ in writes:
        w_.wait()


@functools.cache
def _sc_call():
    return pl.kernel(
        _sc_gather_body,
        out_type=jax.ShapeDtypeStruct((_N, _C * _ED), jnp.float32),
        mesh=plsc.VectorSubcoreMesh(
            core_axis_name="c", subcore_axis_name="s",
            num_cores=_NC, num_subcores=_NS,
        ),
        scratch_types=[
            pltpu.VMEM((2 * _TPW,), jnp.int32),
            pltpu.VMEM((2 * _TPW, _ED), jnp.float32),
            pltpu.SemaphoreType.DMA,
            pltpu.SemaphoreType.DMA,
        ],
    )


def kernel(x, W, b, codebook):
    x2 = x.reshape(_N, _DIN)
    lt, idx0, idx1 = _tc_call(x2, W, b, jnp.asarray(_GUMBEL_T))
    q = _sc_call()(codebook, idx0, idx1)
    logits_out = lt.reshape(_B, _C, _K, _S).transpose(0, 3, 1, 2)
    return (
        q.reshape(_B, _S, _C * _ED),
        logits_out,
    )


# speedup vs baseline: 1.0274x; 1.0274x over previous
import functools

import numpy as np
import jax
import jax.numpy as jnp
from jax import lax
from jax.experimental import pallas as pl
from jax.experimental.pallas import tpu as pltpu
from jax.experimental.pallas import tpu_sc as plsc

_B, _S, _DIN = 4, 512, 768
_C, _K = 2, 320
_ED = 128
_N = _B * _S


def _threefry2x32_np(k0, k1, x0, x1):
    rot = ((13, 15, 26, 6), (17, 29, 16, 24))
    ks = (np.uint32(k0), np.uint32(k1),
          np.uint32(k0) ^ np.uint32(k1) ^ np.uint32(0x1BD11BDA))
    x0 = (x0 + ks[0]).astype(np.uint32)
    x1 = (x1 + ks[1]).astype(np.uint32)
    for i in range(5):
        for r in rot[i % 2]:
            x0 = (x0 + x1).astype(np.uint32)
            x1 = ((x1 << np.uint32(r)) | (x1 >> np.uint32(32 - r))).astype(np.uint32)
            x1 = x1 ^ x0
        x0 = (x0 + ks[(i + 1) % 3]).astype(np.uint32)
        x1 = (x1 + ks[(i + 2) % 3] + np.uint32(i + 1)).astype(np.uint32)
    return x0, x1


def _gumbel_const():
    n = _N * _C * _K
    cnt = np.arange(n, dtype=np.uint32)
    h0, h1 = _threefry2x32_np(0, 42, np.zeros(n, np.uint32), cnt)
    bits = h0 ^ h1
    fb = (bits >> np.uint32(9)) | np.uint32(0x3F800000)
    tiny = np.float32(np.finfo(np.float32).tiny)
    floats = fb.view(np.float32) - np.float32(1.0)
    u = np.maximum(tiny, (floats * (np.float32(1.0) - tiny) + tiny).astype(np.float32))
    return (-np.log(-np.log(u))).astype(np.float32).reshape(_N, _C * _K)


_GUMBEL_T = np.ascontiguousarray(_gumbel_const().T)


def _tc_body(x_ref, w_ref, b_ref, g_ref, lt_ref, idx0_ref, idx1_ref):
    x = x_ref[...]
    w = w_ref[...]
    lt = lax.dot_general(
        w, x, dimension_numbers=(((1,), (1,)), ((), ())),
        preferred_element_type=jnp.float32,
    ) + b_ref[...][:, None]
    lt_ref[0] = lt
    noisy = lt + g_ref[...]
    for c, out_ref in ((0, idx0_ref), (1, idx1_ref)):
        v = noisy[c * _K:(c + 1) * _K, :]
        m = jnp.max(v, axis=0, keepdims=True)
        row = lax.broadcasted_iota(jnp.int32, v.shape, 0)
        out_ref[...] = jnp.min(
            jnp.where(v == m, row, jnp.int32(2**30)), axis=0) + c * _K


_GRID = 4
_CHUNK = _N // _GRID

_tc_call = pl.pallas_call(
    _tc_body,
    grid=(_GRID,),
    in_specs=[
        pl.BlockSpec((_CHUNK, _DIN), lambda i: (i, 0)),
        pl.BlockSpec((_C * _K, _DIN), lambda i: (0, 0)),
        pl.BlockSpec((_C * _K,), lambda i: (0,)),
        pl.BlockSpec((_C * _K, _CHUNK), lambda i: (0, i)),
    ],
    out_specs=(
        pl.BlockSpec((1, _C * _K, _CHUNK),
                     lambda i: (i // (_S // _CHUNK), 0, i % (_S // _CHUNK))),
        pl.BlockSpec((_CHUNK,), lambda i: (i,)),
        pl.BlockSpec((_CHUNK,), lambda i: (i,)),
    ),
    out_shape=(
        jax.ShapeDtypeStruct((_B, _C * _K, _S), jnp.float32),
        jax.ShapeDtypeStruct((_N,), jnp.int32),
        jax.ShapeDtypeStruct((_N,), jnp.int32),
    ),
)

_NC, _NS = 1, 16
_NW = _NC * _NS
_TPW = _N // _NW


_NSEG = 2
_SEG = _TPW // _NSEG


def _sc_gather_body(cb_hbm, idx0_hbm, idx1_hbm, out_hbm, idx_v, rows_v, gsem, wsem):
    wid = lax.axis_index("s") * _NC + lax.axis_index("c")
    base = wid * _TPW
    gathers = []
    for h in range(_NSEG):
        tb = base + h * _SEG
        lo = 2 * h * _SEG
        pltpu.sync_copy(idx0_hbm.at[pl.ds(tb, _SEG)], idx_v.at[pl.ds(lo, _SEG)])
        pltpu.sync_copy(idx1_hbm.at[pl.ds(tb, _SEG)], idx_v.at[pl.ds(lo + _SEG, _SEG)])
        gathers.append(pltpu.async_copy(
            cb_hbm.at[idx_v.at[pl.ds(lo, 2 * _SEG)]],
            rows_v.at[pl.ds(lo, 2 * _SEG)], gsem))
    writes = []
    for h in range(_NSEG):
        gathers[h].wait()
        tb = base + h * _SEG
        lo = 2 * h * _SEG
        writes.append(pltpu.async_copy(
            rows_v.at[pl.ds(lo, _SEG)],
            out_hbm.at[pl.ds(tb, _SEG), pl.ds(0, _ED)], wsem))
        writes.append(pltpu.async_copy(
            rows_v.at[pl.ds(lo + _SEG, _SEG)],
            out_hbm.at[pl.ds(tb, _SEG), pl.ds(_ED, _ED)], wsem))
    for w_ in writes:
        w_.wait()


@functools.cache
def _sc_call():
    return pl.kernel(
        _sc_gather_body,
        out_type=jax.ShapeDtypeStruct((_N, _C * _ED), jnp.float32),
        mesh=plsc.VectorSubcoreMesh(
            core_axis_name="c", subcore_axis_name="s",
            num_cores=_NC, num_subcores=_NS,
        ),
        scratch_types=[
            pltpu.VMEM((2 * _TPW,), jnp.int32),
            pltpu.VMEM((2 * _TPW, _ED), jnp.float32),
            pltpu.SemaphoreType.DMA,
            pltpu.SemaphoreType.DMA,
        ],
    )


def kernel(x, W, b, codebook):
    x2 = x.reshape(_N, _DIN)
    lt, idx0, idx1 = _tc_call(x2, W, b, jnp.asarray(_GUMBEL_T))
    q = _sc_call()(codebook, idx0, idx1)
    logits_out = lt.reshape(_B, _C, _K, _S).transpose(0, 3, 1, 2)
    return (
        q.reshape(_B, _S, _C * _ED),
        logits_out,
    )
